# Initial kernel scaffold; baseline (speedup 1.0000x reference)
import jax
import jax.numpy as jnp
from jax.experimental import pallas as pl


def kernel(x, edge_index, W1, b1, W2, b2, W3, b3):
    raise NotImplementedError("write your pallas kernel here")


# speedup vs baseline: 10.1083x; 10.1083x over previous
import functools

import jax
import jax.numpy as jnp
from jax import lax
from jax.experimental import pallas as pl
from jax.experimental.pallas import tpu as pltpu
from jax.experimental.pallas import tpu_sc as plsc

N = 10000
E = 320000
D_IN = 128
H = 64
C = 40

NPAD = 10240
K = 128
NT = 32
NB = 80
EPAD = NT * NB * K
RPS = NPAD // 16
R = 1024


def _sc_mesh():
    return plsc.VectorSubcoreMesh(core_axis_name="c", subcore_axis_name="s")


_SC_PARAMS = pltpu.CompilerParams(use_tc_tiling_on_sc=False)


def _degree_body(dst_hbm, ones_hbm, z_hbm, out_hbm, idx_d, onesv, acc, *_):
    c = lax.axis_index("c")
    s = lax.axis_index("s")
    w = c * 16 + s
    pltpu.sync_copy(dst_hbm.at[w], idx_d)
    pltpu.sync_copy(ones_hbm, onesv)
    pltpu.sync_copy(z_hbm.at[pl.ds(s * RPS, RPS)], acc.at[pl.ds(s * RPS, RPS)])
    plsc.subcore_barrier()

    def bb(j, carry):
        pltpu.sync_copy(onesv, acc.at[idx_d.at[j]], add=True)
        return carry

    lax.fori_loop(0, NB, bb, 0)
    plsc.subcore_barrier()
    pltpu.sync_copy(acc.at[pl.ds(s * RPS, RPS)], out_hbm.at[c, pl.ds(s * RPS, RPS)])


def _sc_degree(dst_idx, ones16, z16):
    return pl.kernel(
        _degree_body,
        out_type=jax.ShapeDtypeStruct((2, NPAD, 16), jnp.float32),
        mesh=_sc_mesh(),
        scratch_types=[
            pltpu.VMEM((NB, K), jnp.int32),
            pltpu.VMEM((K, 16), jnp.float32),
            pltpu.VMEM_SHARED((NPAD, 16), jnp.float32),
        ],
        compiler_params=_SC_PARAMS,
    )(dst_idx, ones16, z16)


def _edgesum_body(g_hbm, src_hbm, dst_hbm, z_hbm, out_hbm, idx_s, idx_d, rows, acc, *_):
    c = lax.axis_index("c")
    s = lax.axis_index("s")
    w = c * 16 + s
    pltpu.sync_copy(src_hbm.at[w], idx_s)
    pltpu.sync_copy(dst_hbm.at[w], idx_d)
    pltpu.sync_copy(z_hbm.at[pl.ds(s * RPS, RPS)], acc.at[pl.ds(s * RPS, RPS)])
    plsc.subcore_barrier()

    def bb(j, carry):
        pltpu.sync_copy(g_hbm.at[idx_s.at[j]], rows)
        pltpu.sync_copy(rows, acc.at[idx_d.at[j]], add=True)
        return carry

    lax.fori_loop(0, NB, bb, 0)
    plsc.subcore_barrier()
    pltpu.sync_copy(acc.at[pl.ds(s * RPS, RPS)], out_hbm.at[c, pl.ds(s * RPS, RPS)])


def _sc_edgesum(g, src_idx, dst_idx, z64):
    return pl.kernel(
        _edgesum_body,
        out_type=jax.ShapeDtypeStruct((2, NPAD, H), jnp.float32),
        mesh=_sc_mesh(),
        scratch_types=[
            pltpu.VMEM((NB, K), jnp.int32),
            pltpu.VMEM((NB, K), jnp.int32),
            pltpu.VMEM((K, H), jnp.float32),
            pltpu.VMEM_SHARED((NPAD, H), jnp.float32),
        ],
        compiler_params=_SC_PARAMS,
    )(g, src_idx, dst_idx, z64)


def _dinv_from_deg(deg_col, i):
    rows = lax.broadcasted_iota(jnp.int32, (R, 1), 0) + i * R
    return jnp.where(rows < N, lax.rsqrt(deg_col), 0.0)


def _tc_first_body(x_ref, w_ref, degp_ref, m_ref, g_ref, degt_ref):
    i = pl.program_id(0)
    degt = degp_ref[0] + degp_ref[1] + 1.0
    dinv = _dinv_from_deg(degt[:, 0:1], i)
    m = jnp.dot(x_ref[...], w_ref[...], preferred_element_type=jnp.float32)
    m_ref[...] = m
    g_ref[...] = m * dinv
    degt_ref[...] = degt


def _tc_first(xp, w1, degp):
    return pl.pallas_call(
        _tc_first_body,
        grid=(NPAD // R,),
        in_specs=[
            pl.BlockSpec((R, D_IN), lambda i: (i, 0)),
            pl.BlockSpec((D_IN, H), lambda i: (0, 0)),
            pl.BlockSpec((2, R, 16), lambda i: (0, i, 0)),
        ],
        out_specs=[
            pl.BlockSpec((R, H), lambda i: (i, 0)),
            pl.BlockSpec((R, H), lambda i: (i, 0)),
            pl.BlockSpec((R, 16), lambda i: (i, 0)),
        ],
        out_shape=[
            jax.ShapeDtypeStruct((NPAD, H), jnp.float32),
            jax.ShapeDtypeStruct((NPAD, H), jnp.float32),
            jax.ShapeDtypeStruct((NPAD, 16), jnp.float32),
        ],
    )(xp, w1, degp)


def _tc_mid_body(p_ref, m_ref, degt_ref, b_ref, w_ref, mo_ref, go_ref):
    i = pl.program_id(0)
    dinv = _dinv_from_deg(degt_ref[:, 0:1], i)
    o = dinv * (p_ref[0] + p_ref[1]) + (dinv * dinv) * m_ref[...] + b_ref[...]
    h = jnp.maximum(o, 0.0)
    m2 = jnp.dot(h, w_ref[...], preferred_element_type=jnp.float32)
    mo_ref[...] = m2
    go_ref[...] = m2 * dinv


def _tc_mid(p, m, degt, b, w):
    return pl.pallas_call(
        _tc_mid_body,
        grid=(NPAD // R,),
        in_specs=[
            pl.BlockSpec((2, R, H), lambda i: (0, i, 0)),
            pl.BlockSpec((R, H), lambda i: (i, 0)),
            pl.BlockSpec((R, 16), lambda i: (i, 0)),
            pl.BlockSpec((1, H), lambda i: (0, 0)),
            pl.BlockSpec((H, H), lambda i: (0, 0)),
        ],
        out_specs=[
            pl.BlockSpec((R, H), lambda i: (i, 0)),
            pl.BlockSpec((R, H), lambda i: (i, 0)),
        ],
        out_shape=[
            jax.ShapeDtypeStruct((NPAD, H), jnp.float32),
            jax.ShapeDtypeStruct((NPAD, H), jnp.float32),
        ],
    )(p, m, degt, b, w)


def _tc_last_body(p_ref, m_ref, degt_ref, b_ref, o_ref):
    i = pl.program_id(0)
    dinv = _dinv_from_deg(degt_ref[:, 0:1], i)
    o_ref[...] = dinv * (p_ref[0] + p_ref[1]) + (dinv * dinv) * m_ref[...] + b_ref[...]


def _tc_last(p, m, degt, b):
    return pl.pallas_call(
        _tc_last_body,
        grid=(NPAD // R,),
        in_specs=[
            pl.BlockSpec((2, R, H), lambda i: (0, i, 0)),
            pl.BlockSpec((R, H), lambda i: (i, 0)),
            pl.BlockSpec((R, 16), lambda i: (i, 0)),
            pl.BlockSpec((1, H), lambda i: (0, 0)),
        ],
        out_specs=pl.BlockSpec((R, H), lambda i: (i, 0)),
        out_shape=jax.ShapeDtypeStruct((NPAD, H), jnp.float32),
    )(p, m, degt, b)


@jax.jit
def kernel(x, edge_index, W1, b1, W2, b2, W3, b3):
    f32 = jnp.float32
    xp = jnp.zeros((NPAD, D_IN), f32).at[:N].set(x)
    pad = EPAD - E
    src = jnp.concatenate([edge_index[0].astype(jnp.int32), jnp.full((pad,), N, jnp.int32)])
    dst = jnp.concatenate([edge_index[1].astype(jnp.int32), jnp.full((pad,), N, jnp.int32)])
    src_idx = src.reshape(NT, NB, K)
    dst_idx = dst.reshape(NT, NB, K)

    ones16 = jnp.ones((K, 16), f32)
    z16 = jnp.zeros((NPAD, 16), f32)
    z64 = jnp.zeros((NPAD, H), f32)
    w3p = jnp.zeros((H, H), f32).at[:, :C].set(W3)
    b1r = b1.reshape(1, H)
    b2r = b2.reshape(1, H)
    b3r = jnp.zeros((1, H), f32).at[0, :C].set(b3)

    degp = _sc_degree(dst_idx, ones16, z16)
    m1, g1, degt = _tc_first(xp, W1, degp)
    p1 = _sc_edgesum(g1, src_idx, dst_idx, z64)
    m2, g2 = _tc_mid(p1, m1, degt, b1r, W2)
    p2 = _sc_edgesum(g2, src_idx, dst_idx, z64)
    m3, g3 = _tc_mid(p2, m2, degt, b2r, w3p)
    p3 = _sc_edgesum(g3, src_idx, dst_idx, z64)
    out = _tc_last(p3, m3, degt, b3r)
    return out[:N, :C]


# speedup vs baseline: 11.4711x; 1.1348x over previous
import functools

import jax
import jax.numpy as jnp
from jax import lax
from jax.experimental import pallas as pl
from jax.experimental.pallas import tpu as pltpu
from jax.experimental.pallas import tpu_sc as plsc

N = 10000
E = 320000
D_IN = 128
H = 64
C = 40

NPAD = 10240
K = 128
NT = 32
NB = 80
EPAD = NT * NB * K
RPS = NPAD // 16
R = 1024


def _sc_mesh():
    return plsc.VectorSubcoreMesh(core_axis_name="c", subcore_axis_name="s")


_SC_PARAMS = pltpu.CompilerParams(use_tc_tiling_on_sc=False)


def _degree_body(dst_hbm, ones_hbm, z_hbm, out_hbm, idx_d, onesv, acc, *_):
    c = lax.axis_index("c")
    s = lax.axis_index("s")
    w = c * 16 + s
    pltpu.sync_copy(dst_hbm.at[w], idx_d)
    pltpu.sync_copy(ones_hbm, onesv)
    pltpu.sync_copy(z_hbm.at[pl.ds(s * RPS, RPS)], acc.at[pl.ds(s * RPS, RPS)])
    plsc.subcore_barrier()

    def bb(j, carry):
        pltpu.sync_copy(onesv, acc.at[idx_d.at[j]], add=True)
        return carry

    lax.fori_loop(0, NB, bb, 0)
    plsc.subcore_barrier()
    pltpu.sync_copy(acc.at[pl.ds(s * RPS, RPS)], out_hbm.at[c, pl.ds(s * RPS, RPS)])


def _sc_degree(dst_idx, ones16, z16):
    return pl.kernel(
        _degree_body,
        out_type=jax.ShapeDtypeStruct((2, NPAD, 16), jnp.float32),
        mesh=_sc_mesh(),
        scratch_types=[
            pltpu.VMEM((NB, K), jnp.int32),
            pltpu.VMEM((K, 16), jnp.float32),
            pltpu.VMEM_SHARED((NPAD, 16), jnp.float32),
        ],
        compiler_params=_SC_PARAMS,
    )(dst_idx, ones16, z16)


KGRP = 4
NG = NB // KGRP


def _edgesum_body(g_hbm, src_hbm, dst_hbm, z_hbm, out_hbm, idx_s, idx_d, rows, acc,
                  gsem, ssem):
    c = lax.axis_index("c")
    s = lax.axis_index("s")
    w = c * 16 + s
    pltpu.sync_copy(src_hbm.at[w], idx_s)
    pltpu.sync_copy(dst_hbm.at[w], idx_d)
    pltpu.sync_copy(z_hbm.at[pl.ds(s * RPS, RPS)], acc.at[pl.ds(s * RPS, RPS)])
    plsc.subcore_barrier()

    def fire_gathers(grp, set_):
        for b in range(KGRP):
            pltpu.async_copy(g_hbm.at[idx_s.at[grp * KGRP + b]], rows.at[set_, b], gsem)

    fire_gathers(0, 0)

    def bb(grp, carry):
        nxt = grp + 1

        @pl.when(nxt < NG)
        def _():
            fire_gathers(nxt, nxt % 2)

        for b in range(KGRP):
            pltpu.make_async_copy(
                g_hbm.at[idx_s.at[0]], rows.at[grp % 2, b], gsem).wait()
        for b in range(KGRP):
            pltpu.async_copy(
                rows.at[grp % 2, b], acc.at[idx_d.at[grp * KGRP + b]], ssem, add=True)
        for b in range(KGRP):
            pltpu.make_async_copy(
                rows.at[grp % 2, b], acc.at[idx_d.at[0]], ssem).wait()
        return carry

    lax.fori_loop(0, NG, bb, 0)
    plsc.subcore_barrier()
    pltpu.sync_copy(acc.at[pl.ds(s * RPS, RPS)], out_hbm.at[c, pl.ds(s * RPS, RPS)])


def _sc_edgesum(g, src_idx, dst_idx, z64):
    return pl.kernel(
        _edgesum_body,
        out_type=jax.ShapeDtypeStruct((2, NPAD, H), jnp.float32),
        mesh=_sc_mesh(),
        scratch_types=[
            pltpu.VMEM((NB, K), jnp.int32),
            pltpu.VMEM((NB, K), jnp.int32),
            pltpu.VMEM((2, KGRP, K, H), jnp.float32),
            pltpu.VMEM_SHARED((NPAD, H), jnp.float32),
            pltpu.SemaphoreType.DMA,
            pltpu.SemaphoreType.DMA,
        ],
        compiler_params=_SC_PARAMS,
    )(g, src_idx, dst_idx, z64)


def _dinv_from_deg(deg_col, i):
    rows = lax.broadcasted_iota(jnp.int32, (R, 1), 0) + i * R
    return jnp.where(rows < N, lax.rsqrt(deg_col), 0.0)


def _tc_first_body(x_ref, w_ref, degp_ref, m_ref, g_ref, degt_ref):
    i = pl.program_id(0)
    degt = degp_ref[0] + degp_ref[1] + 1.0
    dinv = _dinv_from_deg(degt[:, 0:1], i)
    m = jnp.dot(x_ref[...], w_ref[...], preferred_element_type=jnp.float32)
    m_ref[...] = m
    g_ref[...] = m * dinv
    degt_ref[...] = degt


def _tc_first(xp, w1, degp):
    return pl.pallas_call(
        _tc_first_body,
        grid=(NPAD // R,),
        in_specs=[
            pl.BlockSpec((R, D_IN), lambda i: (i, 0)),
            pl.BlockSpec((D_IN, H), lambda i: (0, 0)),
            pl.BlockSpec((2, R, 16), lambda i: (0, i, 0)),
        ],
        out_specs=[
            pl.BlockSpec((R, H), lambda i: (i, 0)),
            pl.BlockSpec((R, H), lambda i: (i, 0)),
            pl.BlockSpec((R, 16), lambda i: (i, 0)),
        ],
        out_shape=[
            jax.ShapeDtypeStruct((NPAD, H), jnp.float32),
            jax.ShapeDtypeStruct((NPAD, H), jnp.float32),
            jax.ShapeDtypeStruct((NPAD, 16), jnp.float32),
        ],
    )(xp, w1, degp)


def _tc_mid_body(p_ref, m_ref, degt_ref, b_ref, w_ref, mo_ref, go_ref):
    i = pl.program_id(0)
    dinv = _dinv_from_deg(degt_ref[:, 0:1], i)
    o = dinv * (p_ref[0] + p_ref[1]) + (dinv * dinv) * m_ref[...] + b_ref[...]
    h = jnp.maximum(o, 0.0)
    m2 = jnp.dot(h, w_ref[...], preferred_element_type=jnp.float32)
    mo_ref[...] = m2
    go_ref[...] = m2 * dinv


def _tc_mid(p, m, degt, b, w):
    return pl.pallas_call(
        _tc_mid_body,
        grid=(NPAD // R,),
        in_specs=[
            pl.BlockSpec((2, R, H), lambda i: (0, i, 0)),
            pl.BlockSpec((R, H), lambda i: (i, 0)),
            pl.BlockSpec((R, 16), lambda i: (i, 0)),
            pl.BlockSpec((1, H), lambda i: (0, 0)),
            pl.BlockSpec((H, H), lambda i: (0, 0)),
        ],
        out_specs=[
            pl.BlockSpec((R, H), lambda i: (i, 0)),
            pl.BlockSpec((R, H), lambda i: (i, 0)),
        ],
        out_shape=[
            jax.ShapeDtypeStruct((NPAD, H), jnp.float32),
            jax.ShapeDtypeStruct((NPAD, H), jnp.float32),
        ],
    )(p, m, degt, b, w)


def _tc_last_body(p_ref, m_ref, degt_ref, b_ref, o_ref):
    i = pl.program_id(0)
    dinv = _dinv_from_deg(degt_ref[:, 0:1], i)
    o_ref[...] = dinv * (p_ref[0] + p_ref[1]) + (dinv * dinv) * m_ref[...] + b_ref[...]


def _tc_last(p, m, degt, b):
    return pl.pallas_call(
        _tc_last_body,
        grid=(NPAD // R,),
        in_specs=[
            pl.BlockSpec((2, R, H), lambda i: (0, i, 0)),
            pl.BlockSpec((R, H), lambda i: (i, 0)),
            pl.BlockSpec((R, 16), lambda i: (i, 0)),
            pl.BlockSpec((1, H), lambda i: (0, 0)),
        ],
        out_specs=pl.BlockSpec((R, H), lambda i: (i, 0)),
        out_shape=jax.ShapeDtypeStruct((NPAD, H), jnp.float32),
    )(p, m, degt, b)


@jax.jit
def kernel(x, edge_index, W1, b1, W2, b2, W3, b3):
    f32 = jnp.float32
    xp = jnp.zeros((NPAD, D_IN), f32).at[:N].set(x)
    pad = EPAD - E
    src = jnp.concatenate([edge_index[0].astype(jnp.int32), jnp.full((pad,), N, jnp.int32)])
    dst = jnp.concatenate([edge_index[1].astype(jnp.int32), jnp.full((pad,), N, jnp.int32)])
    src_idx = src.reshape(NT, NB, K)
    dst_idx = dst.reshape(NT, NB, K)

    ones16 = jnp.ones((K, 16), f32)
    z16 = jnp.zeros((NPAD, 16), f32)
    z64 = jnp.zeros((NPAD, H), f32)
    w3p = jnp.zeros((H, H), f32).at[:, :C].set(W3)
    b1r = b1.reshape(1, H)
    b2r = b2.reshape(1, H)
    b3r = jnp.zeros((1, H), f32).at[0, :C].set(b3)

    degp = _sc_degree(dst_idx, ones16, z16)
    m1, g1, degt = _tc_first(xp, W1, degp)
    p1 = _sc_edgesum(g1, src_idx, dst_idx, z64)
    m2, g2 = _tc_mid(p1, m1, degt, b1r, W2)
    p2 = _sc_edgesum(g2, src_idx, dst_idx, z64)
    m3, g3 = _tc_mid(p2, m2, degt, b2r, w3p)
    p3 = _sc_edgesum(g3, src_idx, dst_idx, z64)
    out = _tc_last(p3, m3, degt, b3r)
    return out[:N, :C]


# speedup vs baseline: 29.7937x; 2.5973x over previous
import jax
import jax.numpy as jnp
from jax import lax
from jax.experimental import pallas as pl
from jax.experimental.pallas import tpu as pltpu
from jax.experimental.pallas import tpu_sc as plsc

N = 10000
E = 320000
D_IN = 128
H = 64
C = 40

NPAD = 10240
K = 128
HC = H // 2
NBT = 160
EPAD = 16 * NBT * K
RPS = NPAD // 16
R = 1024
NB_DEG = EPAD // (32 * K)


def _sc_mesh():
    return plsc.VectorSubcoreMesh(core_axis_name="c", subcore_axis_name="s")


_SC_PARAMS = pltpu.CompilerParams(use_tc_tiling_on_sc=False)


def _degree_body(dst_hbm, ones_hbm, z_hbm, out_hbm, idx_d, onesv, acc, *_):
    c = lax.axis_index("c")
    s = lax.axis_index("s")
    w = c * 16 + s
    pltpu.sync_copy(dst_hbm.at[w], idx_d)
    pltpu.sync_copy(ones_hbm, onesv)
    pltpu.sync_copy(z_hbm.at[pl.ds(s * RPS, RPS)], acc.at[pl.ds(s * RPS, RPS)])
    plsc.subcore_barrier()

    def bb(j, carry):
        pltpu.sync_copy(onesv, acc.at[idx_d.at[j]], add=True)
        return carry

    lax.fori_loop(0, NB_DEG, bb, 0)
    plsc.subcore_barrier()
    pltpu.sync_copy(acc.at[pl.ds(s * RPS, RPS)], out_hbm.at[c, pl.ds(s * RPS, RPS)])


def _sc_degree(dst_idx, ones16, z16):
    return pl.kernel(
        _degree_body,
        out_type=jax.ShapeDtypeStruct((2, NPAD, 16), jnp.float32),
        mesh=_sc_mesh(),
        scratch_types=[
            pltpu.VMEM((NB_DEG, K), jnp.int32),
            pltpu.VMEM((K, 16), jnp.float32),
            pltpu.VMEM_SHARED((NPAD, 16), jnp.float32),
        ],
        compiler_params=_SC_PARAMS,
    )(dst_idx, ones16, z16)


KGRP = 4
NG = NBT // KGRP


def _edgesum_body(g_hbm, src_hbm, dst_hbm, z_hbm, out_hbm, idx_s, idx_d, rows, gtab,
                  acc, gsem, ssem):
    c = lax.axis_index("c")
    s = lax.axis_index("s")
    pltpu.sync_copy(src_hbm.at[s], idx_s)
    pltpu.sync_copy(dst_hbm.at[s], idx_d)
    pltpu.sync_copy(z_hbm.at[pl.ds(s * RPS, RPS)], acc.at[pl.ds(s * RPS, RPS)])
    pltpu.sync_copy(g_hbm.at[c, pl.ds(s * RPS, RPS)], gtab.at[pl.ds(s * RPS, RPS)])
    plsc.subcore_barrier()

    def fire_gathers(grp, set_):
        for b in range(KGRP):
            pltpu.async_copy(gtab.at[idx_s.at[grp * KGRP + b]], rows.at[set_, b], gsem)

    fire_gathers(0, 0)

    def bb(grp, carry):
        nxt = grp + 1

        @pl.when(nxt < NG)
        def _():
            fire_gathers(nxt, nxt % 2)

        for b in range(KGRP):
            pltpu.make_async_copy(
                gtab.at[idx_s.at[0]], rows.at[grp % 2, b], gsem).wait()
        for b in range(KGRP):
            pltpu.async_copy(
                rows.at[grp % 2, b], acc.at[idx_d.at[grp * KGRP + b]], ssem, add=True)
        for b in range(KGRP):
            pltpu.make_async_copy(
                rows.at[grp % 2, b], acc.at[idx_d.at[0]], ssem).wait()
        return carry

    lax.fori_loop(0, NG, bb, 0)
    plsc.subcore_barrier()
    pltpu.sync_copy(acc.at[pl.ds(s * RPS, RPS)], out_hbm.at[c, pl.ds(s * RPS, RPS)])


def _sc_edgesum(g2, src_idx, dst_idx, z32):
    return pl.kernel(
        _edgesum_body,
        out_type=jax.ShapeDtypeStruct((2, NPAD, HC), jnp.float32),
        mesh=_sc_mesh(),
        scratch_types=[
            pltpu.VMEM((NBT, K), jnp.int32),
            pltpu.VMEM((NBT, K), jnp.int32),
            pltpu.VMEM((2, KGRP, K, HC), jnp.float32),
            pltpu.VMEM_SHARED((NPAD, HC), jnp.float32),
            pltpu.VMEM_SHARED((NPAD, HC), jnp.float32),
            pltpu.SemaphoreType.DMA,
            pltpu.SemaphoreType.DMA,
        ],
        compiler_params=_SC_PARAMS,
    )(g2, src_idx, dst_idx, z32)


def _dinv_from_deg(deg_col, i):
    rows = lax.broadcasted_iota(jnp.int32, (R, 1), 0) + i * R
    return jnp.where(rows < N, lax.rsqrt(deg_col), 0.0)


def _split_cols(gm, go_ref):
    go_ref[0] = gm[:, :HC]
    go_ref[1] = gm[:, HC:]


def _tc_first_body(x_ref, w_ref, degp_ref, m_ref, g_ref, degt_ref):
    i = pl.program_id(0)
    degt = degp_ref[0] + degp_ref[1] + 1.0
    dinv = _dinv_from_deg(degt[:, 0:1], i)
    m = jnp.dot(x_ref[...], w_ref[...], preferred_element_type=jnp.float32)
    m_ref[...] = m
    _split_cols(m * dinv, g_ref)
    degt_ref[...] = degt


def _tc_first(xp, w1, degp):
    return pl.pallas_call(
        _tc_first_body,
        grid=(NPAD // R,),
        in_specs=[
            pl.BlockSpec((R, D_IN), lambda i: (i, 0)),
            pl.BlockSpec((D_IN, H), lambda i: (0, 0)),
            pl.BlockSpec((2, R, 16), lambda i: (0, i, 0)),
        ],
        out_specs=[
            pl.BlockSpec((R, H), lambda i: (i, 0)),
            pl.BlockSpec((2, R, HC), lambda i: (0, i, 0)),
            pl.BlockSpec((R, 16), lambda i: (i, 0)),
        ],
        out_shape=[
            jax.ShapeDtypeStruct((NPAD, H), jnp.float32),
            jax.ShapeDtypeStruct((2, NPAD, HC), jnp.float32),
            jax.ShapeDtypeStruct((NPAD, 16), jnp.float32),
        ],
    )(xp, w1, degp)


def _tc_mid_body(p_ref, m_ref, degt_ref, b_ref, w_ref, mo_ref, go_ref):
    i = pl.program_id(0)
    dinv = _dinv_from_deg(degt_ref[:, 0:1], i)
    p = jnp.concatenate([p_ref[0], p_ref[1]], axis=-1)
    o = dinv * p + (dinv * dinv) * m_ref[...] + b_ref[...]
    h = jnp.maximum(o, 0.0)
    m2 = jnp.dot(h, w_ref[...], preferred_element_type=jnp.float32)
    mo_ref[...] = m2
    _split_cols(m2 * dinv, go_ref)


def _tc_mid(p, m, degt, b, w):
    return pl.pallas_call(
        _tc_mid_body,
        grid=(NPAD // R,),
        in_specs=[
            pl.BlockSpec((2, R, HC), lambda i: (0, i, 0)),
            pl.BlockSpec((R, H), lambda i: (i, 0)),
            pl.BlockSpec((R, 16), lambda i: (i, 0)),
            pl.BlockSpec((1, H), lambda i: (0, 0)),
            pl.BlockSpec((H, H), lambda i: (0, 0)),
        ],
        out_specs=[
            pl.BlockSpec((R, H), lambda i: (i, 0)),
            pl.BlockSpec((2, R, HC), lambda i: (0, i, 0)),
        ],
        out_shape=[
            jax.ShapeDtypeStruct((NPAD, H), jnp.float32),
            jax.ShapeDtypeStruct((2, NPAD, HC), jnp.float32),
        ],
    )(p, m, degt, b, w)


def _tc_last_body(p_ref, m_ref, degt_ref, b_ref, o_ref):
    i = pl.program_id(0)
    dinv = _dinv_from_deg(degt_ref[:, 0:1], i)
    p = jnp.concatenate([p_ref[0], p_ref[1]], axis=-1)
    o_ref[...] = dinv * p + (dinv * dinv) * m_ref[...] + b_ref[...]


def _tc_last(p, m, degt, b):
    return pl.pallas_call(
        _tc_last_body,
        grid=(NPAD // R,),
        in_specs=[
            pl.BlockSpec((2, R, HC), lambda i: (0, i, 0)),
            pl.BlockSpec((R, H), lambda i: (i, 0)),
            pl.BlockSpec((R, 16), lambda i: (i, 0)),
            pl.BlockSpec((1, H), lambda i: (0, 0)),
        ],
        out_specs=pl.BlockSpec((R, H), lambda i: (i, 0)),
        out_shape=jax.ShapeDtypeStruct((NPAD, H), jnp.float32),
    )(p, m, degt, b)


@jax.jit
def kernel(x, edge_index, W1, b1, W2, b2, W3, b3):
    f32 = jnp.float32
    xp = jnp.zeros((NPAD, D_IN), f32).at[:N].set(x)
    pad = EPAD - E
    src = jnp.concatenate([edge_index[0].astype(jnp.int32), jnp.full((pad,), N, jnp.int32)])
    dst = jnp.concatenate([edge_index[1].astype(jnp.int32), jnp.full((pad,), N, jnp.int32)])
    src_idx = src.reshape(16, NBT, K)
    dst_idx = dst.reshape(16, NBT, K)
    dst_idx_deg = dst.reshape(32, NB_DEG, K)

    ones16 = jnp.ones((K, 16), f32)
    z16 = jnp.zeros((NPAD, 16), f32)
    z32 = jnp.zeros((NPAD, HC), f32)
    w3p = jnp.zeros((H, H), f32).at[:, :C].set(W3)
    b1r = b1.reshape(1, H)
    b2r = b2.reshape(1, H)
    b3r = jnp.zeros((1, H), f32).at[0, :C].set(b3)

    degp = _sc_degree(dst_idx_deg, ones16, z16)
    m1, g1, degt = _tc_first(xp, W1, degp)
    p1 = _sc_edgesum(g1, src_idx, dst_idx, z32)
    m2, g2 = _tc_mid(p1, m1, degt, b1r, W2)
    p2 = _sc_edgesum(g2, src_idx, dst_idx, z32)
    m3, g3 = _tc_mid(p2, m2, degt, b2r, w3p)
    p3 = _sc_edgesum(g3, src_idx, dst_idx, z32)
    out = _tc_last(p3, m3, degt, b3r)
    return out[:N, :C]


# speedup vs baseline: 29.9812x; 1.0063x over previous
import jax
import jax.numpy as jnp
from jax import lax
from jax.experimental import pallas as pl
from jax.experimental.pallas import tpu as pltpu
from jax.experimental.pallas import tpu_sc as plsc

N = 10000
E = 320000
D_IN = 128
H = 64
C = 40

NPAD = 10240
K = 128
HC = H // 2
NBT = 160
EPAD = 16 * NBT * K
RPS = NPAD // 16
R = 1024
NB_DEG = EPAD // (32 * K)


def _sc_mesh():
    return plsc.VectorSubcoreMesh(core_axis_name="c", subcore_axis_name="s")


_SC_PARAMS = pltpu.CompilerParams(use_tc_tiling_on_sc=False)


def _degree_body(dst_hbm, ones_hbm, z_hbm, out_hbm, idx_d, onesv, acc, *_):
    c = lax.axis_index("c")
    s = lax.axis_index("s")
    w = c * 16 + s
    pltpu.sync_copy(dst_hbm.at[w], idx_d)
    pltpu.sync_copy(ones_hbm, onesv)
    pltpu.sync_copy(z_hbm.at[pl.ds(s * RPS, RPS)], acc.at[pl.ds(s * RPS, RPS)])
    plsc.subcore_barrier()

    def bb(j, carry):
        pltpu.sync_copy(onesv, acc.at[idx_d.at[j]], add=True)
        return carry

    lax.fori_loop(0, NB_DEG, bb, 0)
    plsc.subcore_barrier()
    pltpu.sync_copy(acc.at[pl.ds(s * RPS, RPS)], out_hbm.at[c, pl.ds(s * RPS, RPS)])


def _sc_degree(dst_idx, ones16, z16):
    return pl.kernel(
        _degree_body,
        out_type=jax.ShapeDtypeStruct((2, NPAD, 16), jnp.float32),
        mesh=_sc_mesh(),
        scratch_types=[
            pltpu.VMEM((NB_DEG, K), jnp.int32),
            pltpu.VMEM((K, 16), jnp.float32),
            pltpu.VMEM_SHARED((NPAD, 16), jnp.float32),
        ],
        compiler_params=_SC_PARAMS,
    )(dst_idx, ones16, z16)


KGRP = 4
NG = NBT // KGRP


def _edgesum_body(g_hbm, src_hbm, dst_hbm, z_hbm, out_hbm, idx_s, idx_d, rows, gtab,
                  acc, gsem, ssem):
    c = lax.axis_index("c")
    s = lax.axis_index("s")
    pltpu.sync_copy(src_hbm.at[s], idx_s)
    pltpu.sync_copy(dst_hbm.at[s], idx_d)
    pltpu.sync_copy(z_hbm.at[pl.ds(s * RPS, RPS)], acc.at[pl.ds(s * RPS, RPS)])
    pltpu.sync_copy(g_hbm.at[c, pl.ds(s * RPS, RPS)], gtab.at[pl.ds(s * RPS, RPS)])
    plsc.subcore_barrier()

    def fire_gathers(grp, set_):
        for b in range(KGRP):
            pltpu.async_copy(gtab.at[idx_s.at[grp * KGRP + b]], rows.at[set_, b], gsem)

    fire_gathers(0, 0)

    def bb(grp, carry):
        nxt = grp + 1

        @pl.when(nxt < NG)
        def _():
            fire_gathers(nxt, nxt % 2)

        for b in range(KGRP):
            pltpu.make_async_copy(
                gtab.at[idx_s.at[0]], rows.at[grp % 2, b], gsem).wait()
        for b in range(KGRP):
            pltpu.async_copy(
                rows.at[grp % 2, b], acc.at[idx_d.at[grp * KGRP + b]], ssem, add=True)
        for b in range(KGRP):
            pltpu.make_async_copy(
                rows.at[grp % 2, b], acc.at[idx_d.at[0]], ssem).wait()
        return carry

    lax.fori_loop(0, NG, bb, 0)
    plsc.subcore_barrier()
    pltpu.sync_copy(acc.at[pl.ds(s * RPS, RPS)], out_hbm.at[c, pl.ds(s * RPS, RPS)])


def _sc_edgesum(g2, src_idx, dst_idx, z32):
    return pl.kernel(
        _edgesum_body,
        out_type=jax.ShapeDtypeStruct((2, NPAD, HC), jnp.float32),
        mesh=_sc_mesh(),
        scratch_types=[
            pltpu.VMEM((NBT, K), jnp.int32),
            pltpu.VMEM((NBT, K), jnp.int32),
            pltpu.VMEM((2, KGRP, K, HC), jnp.float32),
            pltpu.VMEM_SHARED((NPAD, HC), jnp.float32),
            pltpu.VMEM_SHARED((NPAD, HC), jnp.float32),
            pltpu.SemaphoreType.DMA,
            pltpu.SemaphoreType.DMA,
        ],
        compiler_params=_SC_PARAMS,
    )(g2, src_idx, dst_idx, z32)


def _dinv_from_deg(deg_col, i):
    rows = lax.broadcasted_iota(jnp.int32, (R, 1), 0) + i * R
    return jnp.where(rows < N, lax.rsqrt(deg_col), 0.0)


def _split_cols(gm, go_ref):
    go_ref[0] = gm[:, :HC]
    go_ref[1] = gm[:, HC:]


def _tc_matmul1_body(x_ref, w_ref, m_ref):
    m_ref[...] = jnp.dot(x_ref[...], w_ref[...], preferred_element_type=jnp.float32)


def _tc_matmul1(xp, w1):
    return pl.pallas_call(
        _tc_matmul1_body,
        grid=(NPAD // R,),
        in_specs=[
            pl.BlockSpec((R, D_IN), lambda i: (i, 0)),
            pl.BlockSpec((D_IN, H), lambda i: (0, 0)),
        ],
        out_specs=pl.BlockSpec((R, H), lambda i: (i, 0)),
        out_shape=jax.ShapeDtypeStruct((NPAD, H), jnp.float32),
    )(xp, w1)


def _tc_scale_body(m_ref, degp_ref, g_ref, degt_ref):
    i = pl.program_id(0)
    degt = degp_ref[0] + degp_ref[1] + 1.0
    dinv = _dinv_from_deg(degt[:, 0:1], i)
    _split_cols(m_ref[...] * dinv, g_ref)
    degt_ref[...] = degt


def _tc_scale(m, degp):
    return pl.pallas_call(
        _tc_scale_body,
        grid=(NPAD // R,),
        in_specs=[
            pl.BlockSpec((R, H), lambda i: (i, 0)),
            pl.BlockSpec((2, R, 16), lambda i: (0, i, 0)),
        ],
        out_specs=[
            pl.BlockSpec((2, R, HC), lambda i: (0, i, 0)),
            pl.BlockSpec((R, 16), lambda i: (i, 0)),
        ],
        out_shape=[
            jax.ShapeDtypeStruct((2, NPAD, HC), jnp.float32),
            jax.ShapeDtypeStruct((NPAD, 16), jnp.float32),
        ],
    )(m, degp)


def _tc_mid_body(p_ref, m_ref, degt_ref, b_ref, w_ref, mo_ref, go_ref):
    i = pl.program_id(0)
    dinv = _dinv_from_deg(degt_ref[:, 0:1], i)
    p = jnp.concatenate([p_ref[0], p_ref[1]], axis=-1)
    o = dinv * p + (dinv * dinv) * m_ref[...] + b_ref[...]
    h = jnp.maximum(o, 0.0)
    m2 = jnp.dot(h, w_ref[...], preferred_element_type=jnp.float32)
    mo_ref[...] = m2
    _split_cols(m2 * dinv, go_ref)


def _tc_mid(p, m, degt, b, w):
    return pl.pallas_call(
        _tc_mid_body,
        grid=(NPAD // R,),
        in_specs=[
            pl.BlockSpec((2, R, HC), lambda i: (0, i, 0)),
            pl.BlockSpec((R, H), lambda i: (i, 0)),
            pl.BlockSpec((R, 16), lambda i: (i, 0)),
            pl.BlockSpec((1, H), lambda i: (0, 0)),
            pl.BlockSpec((H, H), lambda i: (0, 0)),
        ],
        out_specs=[
            pl.BlockSpec((R, H), lambda i: (i, 0)),
            pl.BlockSpec((2, R, HC), lambda i: (0, i, 0)),
        ],
        out_shape=[
            jax.ShapeDtypeStruct((NPAD, H), jnp.float32),
            jax.ShapeDtypeStruct((2, NPAD, HC), jnp.float32),
        ],
    )(p, m, degt, b, w)


def _tc_last_body(p_ref, m_ref, degt_ref, b_ref, o_ref):
    i = pl.program_id(0)
    dinv = _dinv_from_deg(degt_ref[:, 0:1], i)
    p = jnp.concatenate([p_ref[0], p_ref[1]], axis=-1)
    o_ref[...] = dinv * p + (dinv * dinv) * m_ref[...] + b_ref[...]


def _tc_last(p, m, degt, b):
    return pl.pallas_call(
        _tc_last_body,
        grid=(NPAD // R,),
        in_specs=[
            pl.BlockSpec((2, R, HC), lambda i: (0, i, 0)),
            pl.BlockSpec((R, H), lambda i: (i, 0)),
            pl.BlockSpec((R, 16), lambda i: (i, 0)),
            pl.BlockSpec((1, H), lambda i: (0, 0)),
        ],
        out_specs=pl.BlockSpec((R, H), lambda i: (i, 0)),
        out_shape=jax.ShapeDtypeStruct((NPAD, H), jnp.float32),
    )(p, m, degt, b)


@jax.jit
def kernel(x, edge_index, W1, b1, W2, b2, W3, b3):
    f32 = jnp.float32
    xp = jnp.zeros((NPAD, D_IN), f32).at[:N].set(x)
    pad = EPAD - E
    src = jnp.concatenate([edge_index[0].astype(jnp.int32), jnp.full((pad,), N, jnp.int32)])
    dst = jnp.concatenate([edge_index[1].astype(jnp.int32), jnp.full((pad,), N, jnp.int32)])
    src_idx = src.reshape(16, NBT, K)
    dst_idx = dst.reshape(16, NBT, K)
    dst_idx_deg = dst.reshape(32, NB_DEG, K)

    ones16 = jnp.ones((K, 16), f32)
    z16 = jnp.zeros((NPAD, 16), f32)
    z32 = jnp.zeros((NPAD, HC), f32)
    w3p = jnp.zeros((H, H), f32).at[:, :C].set(W3)
    b1r = b1.reshape(1, H)
    b2r = b2.reshape(1, H)
    b3r = jnp.zeros((1, H), f32).at[0, :C].set(b3)

    degp = _sc_degree(dst_idx_deg, ones16, z16)
    m1 = _tc_matmul1(xp, W1)
    g1, degt = _tc_scale(m1, degp)
    p1 = _sc_edgesum(g1, src_idx, dst_idx, z32)
    m2, g2 = _tc_mid(p1, m1, degt, b1r, W2)
    p2 = _sc_edgesum(g2, src_idx, dst_idx, z32)
    m3, g3 = _tc_mid(p2, m2, degt, b2r, w3p)
    p3 = _sc_edgesum(g3, src_idx, dst_idx, z32)
    out = _tc_last(p3, m3, degt, b3r)
    return out[:N, :C]
